# Initial kernel scaffold; baseline (speedup 1.0000x reference)
import jax
import jax.numpy as jnp
from jax.experimental import pallas as pl


def kernel(h, edge_index, he, W_pool0, b_pool0, W_self0, W_neigh0, b_neigh0, W_pool1, b_pool1, W_self1, W_neigh1, b_neigh1, W_lin, b_lin):
    raise NotImplementedError("write your pallas kernel here")


# speedup vs baseline: 1.3849x; 1.3849x over previous
import jax
import jax.numpy as jnp
from jax import lax
from jax.experimental import pallas as pl
from jax.experimental.pallas import tpu as pltpu
from jax.experimental.pallas import tpu_sc as plsc

_N = 10000
_D = 128
_E = 320000
_BLK = 1000

_NC = 2
_NS = 16
_NW = _NC * _NS
_R = 320
_NPAD = _NW * _R
_C = 12800
_NG = _C // 16
_NCHUNK = _E // _C


def _mm_relu(x, wt, b):
    def body(x_ref, w_ref, b_ref, o_ref):
        acc = jnp.dot(x_ref[...], w_ref[...], preferred_element_type=jnp.float32)
        o_ref[...] = jax.nn.relu(acc + b_ref[...])

    n = x.shape[0]
    return pl.pallas_call(
        body,
        grid=(n // _BLK,),
        in_specs=[
            pl.BlockSpec((_BLK, _D), lambda i: (i, 0)),
            pl.BlockSpec((_D, _D), lambda i: (0, 0)),
            pl.BlockSpec((1, _D), lambda i: (0, 0)),
        ],
        out_specs=pl.BlockSpec((_BLK, _D), lambda i: (i, 0)),
        out_shape=jax.ShapeDtypeStruct((n, _D), jnp.float32),
    )(x, wt, b.reshape(1, _D))


def _combine_relu(h, hn, wst, wnt, b):
    def body(h_ref, hn_ref, ws_ref, wn_ref, b_ref, o_ref):
        acc = jnp.dot(h_ref[...], ws_ref[...], preferred_element_type=jnp.float32)
        acc += jnp.dot(hn_ref[...], wn_ref[...], preferred_element_type=jnp.float32)
        o_ref[...] = jax.nn.relu(acc + b_ref[...])

    return pl.pallas_call(
        body,
        grid=(_N // _BLK,),
        in_specs=[
            pl.BlockSpec((_BLK, _D), lambda i: (i, 0)),
            pl.BlockSpec((_BLK, _D), lambda i: (i, 0)),
            pl.BlockSpec((_D, _D), lambda i: (0, 0)),
            pl.BlockSpec((_D, _D), lambda i: (0, 0)),
            pl.BlockSpec((1, _D), lambda i: (0, 0)),
        ],
        out_specs=pl.BlockSpec((_BLK, _D), lambda i: (i, 0)),
        out_shape=jax.ShapeDtypeStruct((_N, _D), jnp.float32),
    )(h, hn, wst, wnt, b.reshape(1, _D))


def _mean_head(h, wlt, b):
    def body(h_ref, w_ref, b_ref, o_ref, acc_ref):
        i = pl.program_id(0)

        @pl.when(i == 0)
        def _():
            acc_ref[...] = jnp.zeros_like(acc_ref)

        acc_ref[...] += jnp.sum(h_ref[...], axis=0, keepdims=True)

        @pl.when(i == _N // _BLK - 1)
        def _():
            pooled = acc_ref[...] * (1.0 / _N)
            o_ref[...] = (
                jnp.dot(pooled, w_ref[...], preferred_element_type=jnp.float32)
                + b_ref[...]
            )

    return pl.pallas_call(
        body,
        grid=(_N // _BLK,),
        in_specs=[
            pl.BlockSpec((_BLK, _D), lambda i: (i, 0)),
            pl.BlockSpec((_D, _D), lambda i: (0, 0)),
            pl.BlockSpec((1, _D), lambda i: (0, 0)),
        ],
        out_specs=pl.BlockSpec((1, _D), lambda i: (0, 0)),
        out_shape=jax.ShapeDtypeStruct((1, _D), jnp.float32),
        scratch_shapes=[pltpu.VMEM((1, _D), jnp.float32)],
    )(h, wlt, b.reshape(1, _D))


def _segmax_body(hp, srcg, dstg, hn, acc, srcb, dstb, selsrc, seldst, rows,
                 s0, s1, s2, s3):
    sems = (s0, s1, s2, s3)
    cid = lax.axis_index("c")
    sid = lax.axis_index("s")
    wid = sid * _NC + cid
    base = wid * _R
    iota = lax.iota(jnp.int32, 16)
    zeros16 = jnp.zeros((16,), jnp.float32)

    def zero_body(r, carry):
        acc[pl.ds(r * 16, 16)] = zeros16
        return carry

    lax.fori_loop(0, (_R + 1) * _D // 16, zero_body, 0)

    def chunk_body(c, carry):
        pltpu.sync_copy(srcg.at[pl.ds(c * _C, _C)], srcb)
        pltpu.sync_copy(dstg.at[pl.ds(c * _C, _C)], dstb)

        def scan_body(g, cur):
            dstv = dstb[pl.ds(g * 16, 16)]
            srcv = srcb[pl.ds(g * 16, 16)]
            m = (dstv >= base) & (dstv < base + _R)
            dstl = dstv - base
            prefix = jnp.cumsum(jnp.where(m, 1, 0).astype(jnp.int32))
            pos = cur + prefix - 1
            plsc.store_scatter(selsrc, [pos], srcv, mask=m)
            plsc.store_scatter(seldst, [pos], dstl, mask=m)
            return cur + plsc.all_reduce_population_count(m)

        cur = lax.fori_loop(0, _NG, scan_body, jnp.zeros((16,), jnp.int32))
        n_sel = jnp.max(cur)

        for q in range(5):
            posp = n_sel + iota + q * 16
            padsrc = iota * 577 + wid * 29 + q * 89
            plsc.store_scatter(selsrc, [posp], padsrc)
            plsc.store_scatter(seldst, [posp], jnp.full((16,), _R, jnp.int32))

        nblk = (n_sel + 15) // 16

        for b in range(4):
            @pl.when(b < nblk)
            def _issue0(b=b):
                idxv = selsrc[pl.ds(b * 16, 16)]
                pltpu.async_copy(hp.at[idxv], rows.at[b], sems[b])

        nq = (nblk + 3) // 4

        def drain_body(q, carry):
            for b in range(4):
                g = q * 4 + b

                @pl.when(g < nblk)
                def _proc(b=b, g=g):
                    idxv = selsrc[pl.ds(g * 16, 16)]
                    pltpu.make_async_copy(hp.at[idxv], rows.at[b], sems[b]).wait()
                    for i in range(16):
                        rowsel = plsc.load_gather(
                            seldst, [jnp.full((16,), g * 16 + i, jnp.int32)])
                        abase = rowsel * _D
                        for j in range(8):
                            idxj = abase + (iota + 16 * j)
                            accv = plsc.load_gather(acc, [idxj])
                            rv = rows[b, i, pl.ds(16 * j, 16)]
                            plsc.store_scatter(acc, [idxj],
                                               jnp.maximum(accv, rv))
                    nxt = g + 4

                    @pl.when(nxt < nblk)
                    def _issue():
                        idxn = selsrc[pl.ds(nxt * 16, 16)]
                        pltpu.async_copy(hp.at[idxn], rows.at[b], sems[b])

            return carry

        lax.fori_loop(0, nq, drain_body, 0)
        return carry

    lax.fori_loop(0, _NCHUNK, chunk_body, 0)

    pltpu.sync_copy(acc.at[pl.ds(0, _R * _D)], hn.at[pl.ds(base * _D, _R * _D)])


_segmax_call = pl.kernel(
    _segmax_body,
    out_type=jax.ShapeDtypeStruct((_NPAD * _D,), jnp.float32),
    mesh=plsc.VectorSubcoreMesh(core_axis_name="c", subcore_axis_name="s",
                                num_cores=_NC, num_subcores=_NS),
    compiler_params=pltpu.CompilerParams(needs_layout_passes=False),
    scratch_types=[
        pltpu.VMEM(((_R + 1) * _D,), jnp.float32),
        pltpu.VMEM((_C,), jnp.int32),
        pltpu.VMEM((_C,), jnp.int32),
        pltpu.VMEM((_C + 80,), jnp.int32),
        pltpu.VMEM((_C + 80,), jnp.int32),
        pltpu.VMEM((4, 16, _D), jnp.float32),
        pltpu.SemaphoreType.DMA,
        pltpu.SemaphoreType.DMA,
        pltpu.SemaphoreType.DMA,
        pltpu.SemaphoreType.DMA,
    ],
)


def kernel(h, edge_index, he, W_pool0, b_pool0, W_self0, W_neigh0, b_neigh0,
           W_pool1, b_pool1, W_self1, W_neigh1, b_neigh1, W_lin, b_lin):
    src = edge_index[0]
    dst = edge_index[1]
    params = [
        (W_pool0, b_pool0, W_self0, W_neigh0, b_neigh0),
        (W_pool1, b_pool1, W_self1, W_neigh1, b_neigh1),
    ]
    for (Wp, bp, Ws, Wn, bn) in params:
        hp = _mm_relu(h, Wp.T, bp)
        hn = _segmax_call(hp, src, dst).reshape(_NPAD, _D)[:_N]
        h = _combine_relu(h, hn, Ws.T, Wn.T, bn)
    local_feat = h
    global_feat = _mean_head(local_feat, W_lin.T, b_lin)
    return (local_feat, global_feat)


# speedup vs baseline: 2.1393x; 1.5448x over previous
import jax
import jax.numpy as jnp
from jax import lax
from jax.experimental import pallas as pl
from jax.experimental.pallas import tpu as pltpu
from jax.experimental.pallas import tpu_sc as plsc

_N = 10000
_D = 128
_E = 320000
_BLK = 1000

_NC = 2
_NS = 16
_NW = _NC * _NS
_R = 320
_NPAD = _NW * _R
_C = 12800
_NG = _C // 16
_NCHUNK = _E // _C


def _mm_relu(x, wt, b):
    def body(x_ref, w_ref, b_ref, o_ref):
        acc = jnp.dot(x_ref[...], w_ref[...], preferred_element_type=jnp.float32)
        o_ref[...] = jax.nn.relu(acc + b_ref[...])

    n = x.shape[0]
    return pl.pallas_call(
        body,
        grid=(n // _BLK,),
        in_specs=[
            pl.BlockSpec((_BLK, _D), lambda i: (i, 0)),
            pl.BlockSpec((_D, _D), lambda i: (0, 0)),
            pl.BlockSpec((1, _D), lambda i: (0, 0)),
        ],
        out_specs=pl.BlockSpec((_BLK, _D), lambda i: (i, 0)),
        out_shape=jax.ShapeDtypeStruct((n, _D), jnp.float32),
    )(x, wt, b.reshape(1, _D))


def _combine_relu(h, hn, wst, wnt, b):
    def body(h_ref, hn_ref, ws_ref, wn_ref, b_ref, o_ref):
        acc = jnp.dot(h_ref[...], ws_ref[...], preferred_element_type=jnp.float32)
        acc += jnp.dot(hn_ref[...], wn_ref[...], preferred_element_type=jnp.float32)
        o_ref[...] = jax.nn.relu(acc + b_ref[...])

    return pl.pallas_call(
        body,
        grid=(_N // _BLK,),
        in_specs=[
            pl.BlockSpec((_BLK, _D), lambda i: (i, 0)),
            pl.BlockSpec((_BLK, _D), lambda i: (i, 0)),
            pl.BlockSpec((_D, _D), lambda i: (0, 0)),
            pl.BlockSpec((_D, _D), lambda i: (0, 0)),
            pl.BlockSpec((1, _D), lambda i: (0, 0)),
        ],
        out_specs=pl.BlockSpec((_BLK, _D), lambda i: (i, 0)),
        out_shape=jax.ShapeDtypeStruct((_N, _D), jnp.float32),
    )(h, hn, wst, wnt, b.reshape(1, _D))


def _mean_head(h, wlt, b):
    def body(h_ref, w_ref, b_ref, o_ref, acc_ref):
        i = pl.program_id(0)

        @pl.when(i == 0)
        def _():
            acc_ref[...] = jnp.zeros_like(acc_ref)

        acc_ref[...] += jnp.sum(h_ref[...], axis=0, keepdims=True)

        @pl.when(i == _N // _BLK - 1)
        def _():
            pooled = acc_ref[...] * (1.0 / _N)
            o_ref[...] = (
                jnp.dot(pooled, w_ref[...], preferred_element_type=jnp.float32)
                + b_ref[...]
            )

    return pl.pallas_call(
        body,
        grid=(_N // _BLK,),
        in_specs=[
            pl.BlockSpec((_BLK, _D), lambda i: (i, 0)),
            pl.BlockSpec((_D, _D), lambda i: (0, 0)),
            pl.BlockSpec((1, _D), lambda i: (0, 0)),
        ],
        out_specs=pl.BlockSpec((1, _D), lambda i: (0, 0)),
        out_shape=jax.ShapeDtypeStruct((1, _D), jnp.float32),
        scratch_shapes=[pltpu.VMEM((1, _D), jnp.float32)],
    )(h, wlt, b.reshape(1, _D))


def _segmax_body(hp, srcg, dstg, hn, acc, srcb, dstb, selsrc, seldst, rows,
                 s0, s1, s2, s3):
    sems = (s0, s1, s2, s3)
    cid = lax.axis_index("c")
    sid = lax.axis_index("s")
    wid = sid * _NC + cid
    base = wid * _R
    iota = lax.iota(jnp.int32, 16)
    zeros16 = jnp.zeros((16,), jnp.float32)

    def zero_body(r, carry):
        acc[pl.ds(r * 16, 16)] = zeros16
        return carry

    lax.fori_loop(0, (_R + 1) * _D // 16, zero_body, 0)

    def chunk_body(c, carry):
        pltpu.sync_copy(srcg.at[pl.ds(c * _C, _C)], srcb)
        pltpu.sync_copy(dstg.at[pl.ds(c * _C, _C)], dstb)

        def scan_body(g, cur):
            dstv = dstb[pl.ds(g * 16, 16)]
            srcv = srcb[pl.ds(g * 16, 16)]
            m = (dstv >= base) & (dstv < base + _R)
            dstl = dstv - base
            prefix = jnp.cumsum(jnp.where(m, 1, 0).astype(jnp.int32))
            pos = cur + prefix - 1
            plsc.store_scatter(selsrc, [pos], srcv, mask=m)
            plsc.store_scatter(seldst, [pos], dstl, mask=m)
            return cur + plsc.all_reduce_population_count(m)

        cur = lax.fori_loop(0, _NG, scan_body, jnp.zeros((16,), jnp.int32))
        n_sel = jnp.max(cur)

        for q in range(5):
            posp = n_sel + iota + q * 16
            padsrc = iota * 577 + wid * 29 + q * 89
            plsc.store_scatter(selsrc, [posp], padsrc)
            plsc.store_scatter(seldst, [posp], jnp.full((16,), _R, jnp.int32))

        nblk = (n_sel + 15) // 16

        for b in range(4):
            @pl.when(b < nblk)
            def _issue0(b=b):
                idxv = selsrc[pl.ds(b * 16, 16)]
                pltpu.async_copy(hp.at[idxv], rows.at[b], sems[b])

        nq = (nblk + 3) // 4

        def drain_body(q, carry):
            for b in range(4):
                g = q * 4 + b

                @pl.when(g < nblk)
                def _proc(b=b, g=g):
                    idxv = selsrc[pl.ds(g * 16, 16)]
                    pltpu.make_async_copy(hp.at[idxv], rows.at[b], sems[b]).wait()
                    for i in range(16):
                        rowsel = plsc.load_gather(
                            seldst, [jnp.full((16,), g * 16 + i, jnp.int32)])
                        abase = rowsel * _D
                        idxs = [abase + (iota + 16 * j) for j in range(8)]
                        accs = [plsc.load_gather(acc, [idxs[j]])
                                for j in range(8)]
                        rvs = [rows[b, i, pl.ds(16 * j, 16)] for j in range(8)]
                        news = [jnp.maximum(accs[j], rvs[j]) for j in range(8)]
                        for j in range(8):
                            plsc.store_scatter(acc, [idxs[j]], news[j])
                    nxt = g + 4

                    @pl.when(nxt < nblk)
                    def _issue():
                        idxn = selsrc[pl.ds(nxt * 16, 16)]
                        pltpu.async_copy(hp.at[idxn], rows.at[b], sems[b])

            return carry

        lax.fori_loop(0, nq, drain_body, 0)
        return carry

    lax.fori_loop(0, _NCHUNK, chunk_body, 0)

    pltpu.sync_copy(acc.at[pl.ds(0, _R * _D)], hn.at[pl.ds(base * _D, _R * _D)])


_segmax_call = pl.kernel(
    _segmax_body,
    out_type=jax.ShapeDtypeStruct((_NPAD * _D,), jnp.float32),
    mesh=plsc.VectorSubcoreMesh(core_axis_name="c", subcore_axis_name="s",
                                num_cores=_NC, num_subcores=_NS),
    compiler_params=pltpu.CompilerParams(needs_layout_passes=False),
    scratch_types=[
        pltpu.VMEM(((_R + 1) * _D,), jnp.float32),
        pltpu.VMEM((_C,), jnp.int32),
        pltpu.VMEM((_C,), jnp.int32),
        pltpu.VMEM((_C + 80,), jnp.int32),
        pltpu.VMEM((_C + 80,), jnp.int32),
        pltpu.VMEM((4, 16, _D), jnp.float32),
        pltpu.SemaphoreType.DMA,
        pltpu.SemaphoreType.DMA,
        pltpu.SemaphoreType.DMA,
        pltpu.SemaphoreType.DMA,
    ],
)


def kernel(h, edge_index, he, W_pool0, b_pool0, W_self0, W_neigh0, b_neigh0,
           W_pool1, b_pool1, W_self1, W_neigh1, b_neigh1, W_lin, b_lin):
    src = edge_index[0]
    dst = edge_index[1]
    params = [
        (W_pool0, b_pool0, W_self0, W_neigh0, b_neigh0),
        (W_pool1, b_pool1, W_self1, W_neigh1, b_neigh1),
    ]
    for (Wp, bp, Ws, Wn, bn) in params:
        hp = _mm_relu(h, Wp.T, bp)
        hn = _segmax_call(hp, src, dst).reshape(_NPAD, _D)[:_N]
        h = _combine_relu(h, hn, Ws.T, Wn.T, bn)
    local_feat = h
    global_feat = _mean_head(local_feat, W_lin.T, b_lin)
    return (local_feat, global_feat)


# speedup vs baseline: 2.1607x; 1.0100x over previous
import jax
import jax.numpy as jnp
from jax import lax
from jax.experimental import pallas as pl
from jax.experimental.pallas import tpu as pltpu
from jax.experimental.pallas import tpu_sc as plsc

_N = 10000
_D = 128
_E = 320000
_BLK = 1000

_NC = 2
_NS = 16
_NW = _NC * _NS
_R = 320
_NPAD = _NW * _R
_C = 12800
_NG = _C // 16
_NCHUNK = _E // _C
_SELSZ = _C + 80
_SBLK = 2576


def _mm_relu(x, wt, b):
    def body(x_ref, w_ref, b_ref, o_ref):
        acc = jnp.dot(x_ref[...], w_ref[...], preferred_element_type=jnp.float32)
        o_ref[...] = jax.nn.relu(acc + b_ref[...])

    n = x.shape[0]
    return pl.pallas_call(
        body,
        grid=(n // _BLK,),
        in_specs=[
            pl.BlockSpec((_BLK, _D), lambda i: (i, 0)),
            pl.BlockSpec((_D, _D), lambda i: (0, 0)),
            pl.BlockSpec((1, _D), lambda i: (0, 0)),
        ],
        out_specs=pl.BlockSpec((_BLK, _D), lambda i: (i, 0)),
        out_shape=jax.ShapeDtypeStruct((n, _D), jnp.float32),
    )(x, wt, b.reshape(1, _D))


def _combine_relu(h, hn, wst, wnt, b):
    def body(h_ref, hn_ref, ws_ref, wn_ref, b_ref, o_ref):
        acc = jnp.dot(h_ref[...], ws_ref[...], preferred_element_type=jnp.float32)
        acc += jnp.dot(hn_ref[...], wn_ref[...], preferred_element_type=jnp.float32)
        o_ref[...] = jax.nn.relu(acc + b_ref[...])

    return pl.pallas_call(
        body,
        grid=(_N // _BLK,),
        in_specs=[
            pl.BlockSpec((_BLK, _D), lambda i: (i, 0)),
            pl.BlockSpec((_BLK, _D), lambda i: (i, 0)),
            pl.BlockSpec((_D, _D), lambda i: (0, 0)),
            pl.BlockSpec((_D, _D), lambda i: (0, 0)),
            pl.BlockSpec((1, _D), lambda i: (0, 0)),
        ],
        out_specs=pl.BlockSpec((_BLK, _D), lambda i: (i, 0)),
        out_shape=jax.ShapeDtypeStruct((_N, _D), jnp.float32),
    )(h, hn, wst, wnt, b.reshape(1, _D))


def _mean_head(h, wlt, b):
    def body(h_ref, w_ref, b_ref, o_ref, acc_ref):
        i = pl.program_id(0)

        @pl.when(i == 0)
        def _():
            acc_ref[...] = jnp.zeros_like(acc_ref)

        acc_ref[...] += jnp.sum(h_ref[...], axis=0, keepdims=True)

        @pl.when(i == _N // _BLK - 1)
        def _():
            pooled = acc_ref[...] * (1.0 / _N)
            o_ref[...] = (
                jnp.dot(pooled, w_ref[...], preferred_element_type=jnp.float32)
                + b_ref[...]
            )

    return pl.pallas_call(
        body,
        grid=(_N // _BLK,),
        in_specs=[
            pl.BlockSpec((_BLK, _D), lambda i: (i, 0)),
            pl.BlockSpec((_D, _D), lambda i: (0, 0)),
            pl.BlockSpec((1, _D), lambda i: (0, 0)),
        ],
        out_specs=pl.BlockSpec((1, _D), lambda i: (0, 0)),
        out_shape=jax.ShapeDtypeStruct((1, _D), jnp.float32),
        scratch_shapes=[pltpu.VMEM((1, _D), jnp.float32)],
    )(h, wlt, b.reshape(1, _D))


def _zero_acc(acc):
    zeros16 = jnp.zeros((16,), jnp.float32)

    def zero_body(r, carry):
        acc[pl.ds(r * 16, 16)] = zeros16
        return carry

    lax.fori_loop(0, (_R + 1) * _D // 16, zero_body, 0)


def _emit_drain(hp, selsrc, seldst, soff, n_sel, acc, rows, sems, iota):
    nblk = (n_sel + 15) // 16

    for b in range(4):
        @pl.when(b < nblk)
        def _issue0(b=b):
            idxv = selsrc[pl.ds(soff + b * 16, 16)]
            pltpu.async_copy(hp.at[idxv], rows.at[b], sems[b])

    nq = (nblk + 3) // 4

    def drain_body(q, carry):
        for b in range(4):
            g = q * 4 + b

            @pl.when(g < nblk)
            def _proc(b=b, g=g):
                idxv = selsrc[pl.ds(soff + g * 16, 16)]
                pltpu.make_async_copy(hp.at[idxv], rows.at[b], sems[b]).wait()
                for i in range(16):
                    rowsel = plsc.load_gather(
                        seldst, [jnp.full((16,), soff + g * 16 + i, jnp.int32)])
                    abase = rowsel * _D
                    idxs = [abase + (iota + 16 * j) for j in range(8)]
                    accs = [plsc.load_gather(acc, [idxs[j]]) for j in range(8)]
                    rvs = [rows[b, i, pl.ds(16 * j, 16)] for j in range(8)]
                    news = [jnp.maximum(accs[j], rvs[j]) for j in range(8)]
                    for j in range(8):
                        plsc.store_scatter(acc, [idxs[j]], news[j])
                nxt = g + 4

                @pl.when(nxt < nblk)
                def _issue():
                    idxn = selsrc[pl.ds(soff + nxt * 16, 16)]
                    pltpu.async_copy(hp.at[idxn], rows.at[b], sems[b])

        return carry

    lax.fori_loop(0, nq, drain_body, 0)


def _segmax_full_body(hp, srcg, dstg, hn, selo, seldo, cnto,
                      acc, srcb, dstb, selsrc, seldst, rows, cntv,
                      s0, s1, s2, s3, si0, si1, si2, si3):
    sems = (s0, s1, s2, s3)
    isems = ((si0, si1), (si2, si3))
    cid = lax.axis_index("c")
    sid = lax.axis_index("s")
    wid = sid * _NC + cid
    base = wid * _R
    iota = lax.iota(jnp.int32, 16)

    _zero_acc(acc)

    def issue_idx(c, h):
        pltpu.async_copy(srcg.at[pl.ds(c * _C, _C)],
                         srcb.at[pl.ds(h * _C, _C)], isems[h][0])
        pltpu.async_copy(dstg.at[pl.ds(c * _C, _C)],
                         dstb.at[pl.ds(h * _C, _C)], isems[h][1])

    def wait_idx(c, h):
        pltpu.make_async_copy(srcg.at[pl.ds(c * _C, _C)],
                              srcb.at[pl.ds(h * _C, _C)], isems[h][0]).wait()
        pltpu.make_async_copy(dstg.at[pl.ds(c * _C, _C)],
                              dstb.at[pl.ds(h * _C, _C)], isems[h][1]).wait()

    def do_chunk(c, h):
        off = h * _C

        def scan_body(g, cur):
            dstv = dstb[pl.ds(off + g * 16, 16)]
            srcv = srcb[pl.ds(off + g * 16, 16)]
            m = (dstv >= base) & (dstv < base + _R)
            dstl = dstv - base
            prefix = jnp.cumsum(jnp.where(m, 1, 0).astype(jnp.int32))
            pos = cur + prefix - 1
            plsc.store_scatter(selsrc, [pos], srcv, mask=m)
            plsc.store_scatter(seldst, [pos], dstl, mask=m)
            return cur + plsc.all_reduce_population_count(m)

        cur = lax.fori_loop(0, _NG, scan_body, jnp.zeros((16,), jnp.int32))
        n_sel = jnp.max(cur)

        for q in range(5):
            posp = n_sel + iota + q * 16
            padsrc = iota * 577 + wid * 29 + q * 89
            plsc.store_scatter(selsrc, [posp], padsrc)
            plsc.store_scatter(seldst, [posp], jnp.full((16,), _R, jnp.int32))

        plsc.store_scatter(cntv, [jnp.full((16,), c, jnp.int32)],
                           jnp.full((16,), n_sel, jnp.int32))

        _emit_drain(hp, selsrc, seldst, 0, n_sel, acc, rows, sems, iota)

        slot = (wid * _NCHUNK + c) * _SELSZ
        for t in range(5):
            @pl.when(t * _SBLK < n_sel + 80)
            def _spill(t=t):
                pltpu.sync_copy(selsrc.at[pl.ds(t * _SBLK, _SBLK)],
                                selo.at[pl.ds(slot + t * _SBLK, _SBLK)])
                pltpu.sync_copy(seldst.at[pl.ds(t * _SBLK, _SBLK)],
                                seldo.at[pl.ds(slot + t * _SBLK, _SBLK)])

    issue_idx(0, 0)

    def pair_body(q, carry):
        c0 = 2 * q
        wait_idx(c0, 0)
        issue_idx(c0 + 1, 1)
        do_chunk(c0, 0)
        wait_idx(c0 + 1, 1)
        issue_idx(c0 + 2, 0)
        do_chunk(c0 + 1, 1)
        return carry

    lax.fori_loop(0, (_NCHUNK - 1) // 2, pair_body, 0)
    wait_idx(_NCHUNK - 1, 0)
    do_chunk(_NCHUNK - 1, 0)

    pltpu.sync_copy(cntv, cnto.at[pl.ds(wid * 32, 32)])
    pltpu.sync_copy(acc.at[pl.ds(0, _R * _D)], hn.at[pl.ds(base * _D, _R * _D)])


def _segmax_drain_body(hp, seli, seldi, cnti, hn,
                       acc, selsrc, seldst, rows, cntv,
                       s0, s1, s2, s3, p0, p1, p2, p3):
    sems = (s0, s1, s2, s3)
    psems = ((p0, p1), (p2, p3))
    cid = lax.axis_index("c")
    sid = lax.axis_index("s")
    wid = sid * _NC + cid
    base = wid * _R
    iota = lax.iota(jnp.int32, 16)

    _zero_acc(acc)
    pltpu.sync_copy(cnti.at[pl.ds(wid * 32, 32)], cntv)

    def slot_of(c):
        return (wid * _NCHUNK + c) * _SELSZ

    def issue_b0(c, h):
        slot = slot_of(c)
        pltpu.async_copy(seli.at[pl.ds(slot, _SBLK)],
                         selsrc.at[pl.ds(h * _SELSZ, _SBLK)], psems[h][0])
        pltpu.async_copy(seldi.at[pl.ds(slot, _SBLK)],
                         seldst.at[pl.ds(h * _SELSZ, _SBLK)], psems[h][1])

    def wait_b0(c, h):
        slot = slot_of(c)
        pltpu.make_async_copy(seli.at[pl.ds(slot, _SBLK)],
                              selsrc.at[pl.ds(h * _SELSZ, _SBLK)],
                              psems[h][0]).wait()
        pltpu.make_async_copy(seldi.at[pl.ds(slot, _SBLK)],
                              seldst.at[pl.ds(h * _SELSZ, _SBLK)],
                              psems[h][1]).wait()

    def do_chunk(c, h):
        cv = plsc.load_gather(cntv, [jnp.full((16,), c, jnp.int32)])
        n_sel = jnp.max(cv)
        wait_b0(c, h)
        slot = slot_of(c)
        for t in range(1, 5):
            @pl.when(t * _SBLK < n_sel + 80)
            def _load(t=t):
                pltpu.sync_copy(seli.at[pl.ds(slot + t * _SBLK, _SBLK)],
                                selsrc.at[pl.ds(h * _SELSZ + t * _SBLK, _SBLK)])
                pltpu.sync_copy(seldi.at[pl.ds(slot + t * _SBLK, _SBLK)],
                                seldst.at[pl.ds(h * _SELSZ + t * _SBLK, _SBLK)])

        _emit_drain(hp, selsrc, seldst, h * _SELSZ, n_sel, acc, rows, sems,
                    iota)

    issue_b0(0, 0)
    issue_b0(1, 1)

    def pair_body(q, carry):
        c0 = 2 * q
        do_chunk(c0, 0)
        issue_b0(c0 + 2, 0)
        do_chunk(c0 + 1, 1)

        @pl.when(c0 + 3 < _NCHUNK)
        def _pf():
            issue_b0(c0 + 3, 1)

        return carry

    lax.fori_loop(0, (_NCHUNK - 1) // 2, pair_body, 0)
    do_chunk(_NCHUNK - 1, 0)

    pltpu.sync_copy(acc.at[pl.ds(0, _R * _D)], hn.at[pl.ds(base * _D, _R * _D)])


_sc_mesh = plsc.VectorSubcoreMesh(core_axis_name="c", subcore_axis_name="s",
                                  num_cores=_NC, num_subcores=_NS)
_sc_params = pltpu.CompilerParams(needs_layout_passes=False)

_segmax_full = pl.kernel(
    _segmax_full_body,
    out_type=(
        jax.ShapeDtypeStruct((_NPAD * _D,), jnp.float32),
        jax.ShapeDtypeStruct((_NW * _NCHUNK * _SELSZ,), jnp.int32),
        jax.ShapeDtypeStruct((_NW * _NCHUNK * _SELSZ,), jnp.int32),
        jax.ShapeDtypeStruct((_NW * 32,), jnp.int32),
    ),
    mesh=_sc_mesh,
    compiler_params=_sc_params,
    scratch_types=[
        pltpu.VMEM(((_R + 1) * _D,), jnp.float32),
        pltpu.VMEM((2 * _C,), jnp.int32),
        pltpu.VMEM((2 * _C,), jnp.int32),
        pltpu.VMEM((_SELSZ,), jnp.int32),
        pltpu.VMEM((_SELSZ,), jnp.int32),
        pltpu.VMEM((4, 16, _D), jnp.float32),
        pltpu.VMEM((32,), jnp.int32),
        pltpu.SemaphoreType.DMA, pltpu.SemaphoreType.DMA,
        pltpu.SemaphoreType.DMA, pltpu.SemaphoreType.DMA,
        pltpu.SemaphoreType.DMA, pltpu.SemaphoreType.DMA,
        pltpu.SemaphoreType.DMA, pltpu.SemaphoreType.DMA,
    ],
)

_segmax_drain = pl.kernel(
    _segmax_drain_body,
    out_type=jax.ShapeDtypeStruct((_NPAD * _D,), jnp.float32),
    mesh=_sc_mesh,
    compiler_params=_sc_params,
    scratch_types=[
        pltpu.VMEM(((_R + 1) * _D,), jnp.float32),
        pltpu.VMEM((2 * _SELSZ,), jnp.int32),
        pltpu.VMEM((2 * _SELSZ,), jnp.int32),
        pltpu.VMEM((4, 16, _D), jnp.float32),
        pltpu.VMEM((32,), jnp.int32),
        pltpu.SemaphoreType.DMA, pltpu.SemaphoreType.DMA,
        pltpu.SemaphoreType.DMA, pltpu.SemaphoreType.DMA,
        pltpu.SemaphoreType.DMA, pltpu.SemaphoreType.DMA,
        pltpu.SemaphoreType.DMA, pltpu.SemaphoreType.DMA,
    ],
)


def kernel(h, edge_index, he, W_pool0, b_pool0, W_self0, W_neigh0, b_neigh0,
           W_pool1, b_pool1, W_self1, W_neigh1, b_neigh1, W_lin, b_lin):
    src = edge_index[0]
    dst = edge_index[1]

    hp0 = _mm_relu(h, W_pool0.T, b_pool0)
    hn0, selo, seldo, cnts = _segmax_full(hp0, src, dst)
    hn0 = hn0.reshape(_NPAD, _D)[:_N]
    h1 = _combine_relu(h, hn0, W_self0.T, W_neigh0.T, b_neigh0)

    hp1 = _mm_relu(h1, W_pool1.T, b_pool1)
    hn1 = _segmax_drain(hp1, selo, seldo, cnts).reshape(_NPAD, _D)[:_N]
    h2 = _combine_relu(h1, hn1, W_self1.T, W_neigh1.T, b_neigh1)

    global_feat = _mean_head(h2, W_lin.T, b_lin)
    return (h2, global_feat)


# speedup vs baseline: 2.9320x; 1.3570x over previous
import jax
import jax.numpy as jnp
from jax import lax
from jax.experimental import pallas as pl
from jax.experimental.pallas import tpu as pltpu
from jax.experimental.pallas import tpu_sc as plsc

_N = 10000
_D = 128
_E = 320000
_BLK = 1000

_NC = 2
_NS = 16
_NW = _NC * _NS
_R = 320
_NPAD = _NW * _R
_C = 12800
_NG = _C // 16
_NCHUNK = _E // _C
_SELSZ = _C + 80
_SBLK = 2576
_G = 32


def _mm_relu(x, wt, b):
    def body(x_ref, w_ref, b_ref, o_ref):
        acc = jnp.dot(x_ref[...], w_ref[...], preferred_element_type=jnp.float32)
        o_ref[...] = jax.nn.relu(acc + b_ref[...])

    n = x.shape[0]
    return pl.pallas_call(
        body,
        grid=(n // _BLK,),
        in_specs=[
            pl.BlockSpec((_BLK, _D), lambda i: (i, 0)),
            pl.BlockSpec((_D, _D), lambda i: (0, 0)),
            pl.BlockSpec((1, _D), lambda i: (0, 0)),
        ],
        out_specs=pl.BlockSpec((_BLK, _D), lambda i: (i, 0)),
        out_shape=jax.ShapeDtypeStruct((n, _D), jnp.float32),
    )(x, wt, b.reshape(1, _D))


def _combine_relu(h, hn, wst, wnt, b):
    def body(h_ref, hn_ref, ws_ref, wn_ref, b_ref, o_ref):
        acc = jnp.dot(h_ref[...], ws_ref[...], preferred_element_type=jnp.float32)
        acc += jnp.dot(hn_ref[...], wn_ref[...], preferred_element_type=jnp.float32)
        o_ref[...] = jax.nn.relu(acc + b_ref[...])

    return pl.pallas_call(
        body,
        grid=(_N // _BLK,),
        in_specs=[
            pl.BlockSpec((_BLK, _D), lambda i: (i, 0)),
            pl.BlockSpec((_BLK, _D), lambda i: (i, 0)),
            pl.BlockSpec((_D, _D), lambda i: (0, 0)),
            pl.BlockSpec((_D, _D), lambda i: (0, 0)),
            pl.BlockSpec((1, _D), lambda i: (0, 0)),
        ],
        out_specs=pl.BlockSpec((_BLK, _D), lambda i: (i, 0)),
        out_shape=jax.ShapeDtypeStruct((_N, _D), jnp.float32),
    )(h, hn, wst, wnt, b.reshape(1, _D))


def _mean_head(h, wlt, b):
    def body(h_ref, w_ref, b_ref, o_ref, acc_ref):
        i = pl.program_id(0)

        @pl.when(i == 0)
        def _():
            acc_ref[...] = jnp.zeros_like(acc_ref)

        acc_ref[...] += jnp.sum(h_ref[...], axis=0, keepdims=True)

        @pl.when(i == _N // _BLK - 1)
        def _():
            pooled = acc_ref[...] * (1.0 / _N)
            o_ref[...] = (
                jnp.dot(pooled, w_ref[...], preferred_element_type=jnp.float32)
                + b_ref[...]
            )

    return pl.pallas_call(
        body,
        grid=(_N // _BLK,),
        in_specs=[
            pl.BlockSpec((_BLK, _D), lambda i: (i, 0)),
            pl.BlockSpec((_D, _D), lambda i: (0, 0)),
            pl.BlockSpec((1, _D), lambda i: (0, 0)),
        ],
        out_specs=pl.BlockSpec((1, _D), lambda i: (0, 0)),
        out_shape=jax.ShapeDtypeStruct((1, _D), jnp.float32),
        scratch_shapes=[pltpu.VMEM((1, _D), jnp.float32)],
    )(h, wlt, b.reshape(1, _D))


def _zero_acc(acc):
    zeros16 = jnp.zeros((16,), jnp.float32)

    def zero_body(r, carry):
        acc[pl.ds(r * 16, 16)] = zeros16
        return carry

    lax.fori_loop(0, (_R + 1) * _D // 16, zero_body, 0)


def _emit_drain(hp, selsrc, seldst, soff, n_sel, acc, rows, sems, iota):
    nblk = (n_sel + _G - 1) // _G

    for b in range(4):
        @pl.when(b < nblk)
        def _issue0(b=b):
            pltpu.async_copy(hp.at[selsrc.at[pl.ds(soff + b * _G, _G)]],
                             rows.at[b], sems[b])

    nq = (nblk + 3) // 4

    def drain_body(q, carry):
        for b in range(4):
            g = q * 4 + b

            @pl.when(g < nblk)
            def _proc(b=b, g=g):
                pltpu.make_async_copy(
                    hp.at[selsrc.at[pl.ds(soff + g * _G, _G)]],
                    rows.at[b], sems[b]).wait()

                def sub_body(sub, carry):
                    for i in range(16):
                        rowsel = plsc.load_gather(
                            seldst,
                            [jnp.full((16,), soff + g * _G + sub * 16 + i,
                                      jnp.int32)])
                        abase = rowsel * _D
                        idxs = [abase + (iota + 16 * j) for j in range(8)]
                        accs = [plsc.load_gather(acc, [idxs[j]])
                                for j in range(8)]
                        rvs = [rows[b, sub * 16 + i, pl.ds(16 * j, 16)]
                               for j in range(8)]
                        news = [jnp.maximum(accs[j], rvs[j])
                                for j in range(8)]
                        for j in range(8):
                            plsc.store_scatter(acc, [idxs[j]], news[j])
                    return carry

                lax.fori_loop(0, _G // 16, sub_body, 0)
                nxt = g + 4

                @pl.when(nxt < nblk)
                def _issue():
                    pltpu.async_copy(
                        hp.at[selsrc.at[pl.ds(soff + nxt * _G, _G)]],
                        rows.at[b], sems[b])

        return carry

    lax.fori_loop(0, nq, drain_body, 0)


def _segmax_full_body(hp, srcg, dstg, hn, selo, seldo, cnto,
                      acc, srcb, dstb, selsrc, seldst, rows, cntv,
                      s0, s1, s2, s3):
    sems = (s0, s1, s2, s3)
    cid = lax.axis_index("c")
    sid = lax.axis_index("s")
    wid = sid * _NC + cid
    base = wid * _R
    iota = lax.iota(jnp.int32, 16)

    _zero_acc(acc)

    def do_chunk(c, carry):
        pltpu.sync_copy(srcg.at[pl.ds(c * _C, _C)], srcb)
        pltpu.sync_copy(dstg.at[pl.ds(c * _C, _C)], dstb)

        def scan_body(g, cur):
            dstv = dstb[pl.ds(g * 16, 16)]
            srcv = srcb[pl.ds(g * 16, 16)]
            m = (dstv >= base) & (dstv < base + _R)
            dstl = dstv - base
            prefix = jnp.cumsum(jnp.where(m, 1, 0).astype(jnp.int32))
            pos = cur + prefix - 1
            plsc.store_scatter(selsrc, [pos], srcv, mask=m)
            plsc.store_scatter(seldst, [pos], dstl, mask=m)
            return cur + plsc.all_reduce_population_count(m)

        cur = lax.fori_loop(0, _NG, scan_body, jnp.zeros((16,), jnp.int32))
        n_sel = jnp.max(cur)

        for q in range(5):
            posp = n_sel + iota + q * 16
            padsrc = iota * 577 + wid * 29 + q * 89
            plsc.store_scatter(selsrc, [posp], padsrc)
            plsc.store_scatter(seldst, [posp], jnp.full((16,), _R, jnp.int32))

        plsc.store_scatter(cntv, [jnp.full((16,), c, jnp.int32)],
                           jnp.full((16,), n_sel, jnp.int32))

        _emit_drain(hp, selsrc, seldst, 0, n_sel, acc, rows, sems, iota)

        slot = (wid * _NCHUNK + c) * _SELSZ
        for t in range(5):
            @pl.when(t * _SBLK < n_sel + 80)
            def _spill(t=t):
                pltpu.sync_copy(selsrc.at[pl.ds(t * _SBLK, _SBLK)],
                                selo.at[pl.ds(slot + t * _SBLK, _SBLK)])
                pltpu.sync_copy(seldst.at[pl.ds(t * _SBLK, _SBLK)],
                                seldo.at[pl.ds(slot + t * _SBLK, _SBLK)])
        return carry

    lax.fori_loop(0, _NCHUNK, do_chunk, 0)

    pltpu.sync_copy(cntv, cnto.at[pl.ds(wid * 32, 32)])
    pltpu.sync_copy(acc.at[pl.ds(0, _R * _D)], hn.at[pl.ds(base * _D, _R * _D)])


def _segmax_drain_body(hp, seli, seldi, cnti, hn,
                       acc, selsrc, seldst, rows, cntv,
                       s0, s1, s2, s3, p0, p1, p2, p3):
    sems = (s0, s1, s2, s3)
    psems = ((p0, p1), (p2, p3))
    cid = lax.axis_index("c")
    sid = lax.axis_index("s")
    wid = sid * _NC + cid
    base = wid * _R
    iota = lax.iota(jnp.int32, 16)

    _zero_acc(acc)
    pltpu.sync_copy(cnti.at[pl.ds(wid * 32, 32)], cntv)

    def slot_of(c):
        return (wid * _NCHUNK + c) * _SELSZ

    def issue_b0(c, h):
        slot = slot_of(c)
        pltpu.async_copy(seli.at[pl.ds(slot, _SBLK)],
                         selsrc.at[pl.ds(h * _SELSZ, _SBLK)], psems[h][0])
        pltpu.async_copy(seldi.at[pl.ds(slot, _SBLK)],
                         seldst.at[pl.ds(h * _SELSZ, _SBLK)], psems[h][1])

    def wait_b0(c, h):
        slot = slot_of(c)
        pltpu.make_async_copy(seli.at[pl.ds(slot, _SBLK)],
                              selsrc.at[pl.ds(h * _SELSZ, _SBLK)],
                              psems[h][0]).wait()
        pltpu.make_async_copy(seldi.at[pl.ds(slot, _SBLK)],
                              seldst.at[pl.ds(h * _SELSZ, _SBLK)],
                              psems[h][1]).wait()

    def do_chunk(c, h):
        cv = plsc.load_gather(cntv, [jnp.full((16,), c, jnp.int32)])
        n_sel = jnp.max(cv)
        wait_b0(c, h)
        slot = slot_of(c)
        for t in range(1, 5):
            @pl.when(t * _SBLK < n_sel + 80)
            def _load(t=t):
                pltpu.sync_copy(seli.at[pl.ds(slot + t * _SBLK, _SBLK)],
                                selsrc.at[pl.ds(h * _SELSZ + t * _SBLK, _SBLK)])
                pltpu.sync_copy(seldi.at[pl.ds(slot + t * _SBLK, _SBLK)],
                                seldst.at[pl.ds(h * _SELSZ + t * _SBLK, _SBLK)])

        _emit_drain(hp, selsrc, seldst, h * _SELSZ, n_sel, acc, rows, sems,
                    iota)

    issue_b0(0, 0)
    issue_b0(1, 1)

    def pair_body(q, carry):
        c0 = 2 * q
        do_chunk(c0, 0)
        issue_b0(c0 + 2, 0)
        do_chunk(c0 + 1, 1)

        @pl.when(c0 + 3 < _NCHUNK)
        def _pf():
            issue_b0(c0 + 3, 1)

        return carry

    lax.fori_loop(0, (_NCHUNK - 1) // 2, pair_body, 0)
    do_chunk(_NCHUNK - 1, 0)

    pltpu.sync_copy(acc.at[pl.ds(0, _R * _D)], hn.at[pl.ds(base * _D, _R * _D)])


_sc_mesh = plsc.VectorSubcoreMesh(core_axis_name="c", subcore_axis_name="s",
                                  num_cores=_NC, num_subcores=_NS)
_sc_params = pltpu.CompilerParams(needs_layout_passes=False)

_segmax_full = pl.kernel(
    _segmax_full_body,
    out_type=(
        jax.ShapeDtypeStruct((_NPAD * _D,), jnp.float32),
        jax.ShapeDtypeStruct((_NW * _NCHUNK * _SELSZ,), jnp.int32),
        jax.ShapeDtypeStruct((_NW * _NCHUNK * _SELSZ,), jnp.int32),
        jax.ShapeDtypeStruct((_NW * 32,), jnp.int32),
    ),
    mesh=_sc_mesh,
    compiler_params=_sc_params,
    scratch_types=[
        pltpu.VMEM(((_R + 1) * _D,), jnp.float32),
        pltpu.VMEM((_C,), jnp.int32),
        pltpu.VMEM((_C,), jnp.int32),
        pltpu.VMEM((_SELSZ,), jnp.int32),
        pltpu.VMEM((_SELSZ,), jnp.int32),
        pltpu.VMEM((4, _G, _D), jnp.float32),
        pltpu.VMEM((32,), jnp.int32),
        pltpu.SemaphoreType.DMA, pltpu.SemaphoreType.DMA,
        pltpu.SemaphoreType.DMA, pltpu.SemaphoreType.DMA,
    ],
)

_segmax_drain = pl.kernel(
    _segmax_drain_body,
    out_type=jax.ShapeDtypeStruct((_NPAD * _D,), jnp.float32),
    mesh=_sc_mesh,
    compiler_params=_sc_params,
    scratch_types=[
        pltpu.VMEM(((_R + 1) * _D,), jnp.float32),
        pltpu.VMEM((2 * _SELSZ,), jnp.int32),
        pltpu.VMEM((2 * _SELSZ,), jnp.int32),
        pltpu.VMEM((4, _G, _D), jnp.float32),
        pltpu.VMEM((32,), jnp.int32),
        pltpu.SemaphoreType.DMA, pltpu.SemaphoreType.DMA,
        pltpu.SemaphoreType.DMA, pltpu.SemaphoreType.DMA,
        pltpu.SemaphoreType.DMA, pltpu.SemaphoreType.DMA,
        pltpu.SemaphoreType.DMA, pltpu.SemaphoreType.DMA,
    ],
)


def kernel(h, edge_index, he, W_pool0, b_pool0, W_self0, W_neigh0, b_neigh0,
           W_pool1, b_pool1, W_self1, W_neigh1, b_neigh1, W_lin, b_lin):
    src = edge_index[0]
    dst = edge_index[1]

    hp0 = _mm_relu(h, W_pool0.T, b_pool0)
    hn0, selo, seldo, cnts = _segmax_full(hp0, src, dst)
    hn0 = hn0.reshape(_NPAD, _D)[:_N]
    h1 = _combine_relu(h, hn0, W_self0.T, W_neigh0.T, b_neigh0)

    hp1 = _mm_relu(h1, W_pool1.T, b_pool1)
    hn1 = _segmax_drain(hp1, selo, seldo, cnts).reshape(_NPAD, _D)[:_N]
    h2 = _combine_relu(h1, hn1, W_self1.T, W_neigh1.T, b_neigh1)

    global_feat = _mean_head(h2, W_lin.T, b_lin)
    return (h2, global_feat)


# speedup vs baseline: 2.9758x; 1.0149x over previous
import jax
import jax.numpy as jnp
from jax import lax
from jax.experimental import pallas as pl
from jax.experimental.pallas import tpu as pltpu
from jax.experimental.pallas import tpu_sc as plsc

_N = 10000
_D = 128
_E = 320000
_BLK = 1000

_NC = 2
_NS = 16
_NW = _NC * _NS
_R = 320
_NPAD = _NW * _R
_C = 12800
_NG = _C // 16
_NCHUNK = _E // _C
_SELSZ = _C + 80
_SBLK = 2576
_G = 32


def _mm_relu(x, wt, b):
    def body(x_ref, w_ref, b_ref, o_ref):
        acc = jnp.dot(x_ref[...], w_ref[...], preferred_element_type=jnp.float32)
        o_ref[...] = jax.nn.relu(acc + b_ref[...])

    n = x.shape[0]
    return pl.pallas_call(
        body,
        grid=(n // _BLK,),
        in_specs=[
            pl.BlockSpec((_BLK, _D), lambda i: (i, 0)),
            pl.BlockSpec((_D, _D), lambda i: (0, 0)),
            pl.BlockSpec((1, _D), lambda i: (0, 0)),
        ],
        out_specs=pl.BlockSpec((_BLK, _D), lambda i: (i, 0)),
        out_shape=jax.ShapeDtypeStruct((n, _D), jnp.float32),
    )(x, wt, b.reshape(1, _D))


def _combine_relu(h, hn, wst, wnt, b):
    def body(h_ref, hn_ref, ws_ref, wn_ref, b_ref, o_ref):
        acc = jnp.dot(h_ref[...], ws_ref[...], preferred_element_type=jnp.float32)
        acc += jnp.dot(hn_ref[...], wn_ref[...], preferred_element_type=jnp.float32)
        o_ref[...] = jax.nn.relu(acc + b_ref[...])

    return pl.pallas_call(
        body,
        grid=(_N // _BLK,),
        in_specs=[
            pl.BlockSpec((_BLK, _D), lambda i: (i, 0)),
            pl.BlockSpec((_BLK, _D), lambda i: (i, 0)),
            pl.BlockSpec((_D, _D), lambda i: (0, 0)),
            pl.BlockSpec((_D, _D), lambda i: (0, 0)),
            pl.BlockSpec((1, _D), lambda i: (0, 0)),
        ],
        out_specs=pl.BlockSpec((_BLK, _D), lambda i: (i, 0)),
        out_shape=jax.ShapeDtypeStruct((_N, _D), jnp.float32),
    )(h, hn, wst, wnt, b.reshape(1, _D))


def _mean_head(h, wlt, b):
    def body(h_ref, w_ref, b_ref, o_ref, acc_ref):
        i = pl.program_id(0)

        @pl.when(i == 0)
        def _():
            acc_ref[...] = jnp.zeros_like(acc_ref)

        acc_ref[...] += jnp.sum(h_ref[...], axis=0, keepdims=True)

        @pl.when(i == _N // _BLK - 1)
        def _():
            pooled = acc_ref[...] * (1.0 / _N)
            o_ref[...] = (
                jnp.dot(pooled, w_ref[...], preferred_element_type=jnp.float32)
                + b_ref[...]
            )

    return pl.pallas_call(
        body,
        grid=(_N // _BLK,),
        in_specs=[
            pl.BlockSpec((_BLK, _D), lambda i: (i, 0)),
            pl.BlockSpec((_D, _D), lambda i: (0, 0)),
            pl.BlockSpec((1, _D), lambda i: (0, 0)),
        ],
        out_specs=pl.BlockSpec((1, _D), lambda i: (0, 0)),
        out_shape=jax.ShapeDtypeStruct((1, _D), jnp.float32),
        scratch_shapes=[pltpu.VMEM((1, _D), jnp.float32)],
    )(h, wlt, b.reshape(1, _D))


def _take16(x, idx):
    dn = lax.GatherDimensionNumbers(offset_dims=(), collapsed_slice_dims=(0,),
                                    start_index_map=(0,))
    return lax.gather(x, idx.reshape(16, 1), dn, (1,),
                      mode=lax.GatherScatterMode.PROMISE_IN_BOUNDS)


def _zero_acc(acc):
    zeros16 = jnp.zeros((16,), jnp.float32)

    def zero_body(r, carry):
        acc[pl.ds(r * 16, 16)] = zeros16
        return carry

    lax.fori_loop(0, (_R + 1) * _D // 16, zero_body, 0)


def _emit_drain(hp, selsrc, seldst, soff, n_sel, acc, rows, sems, iota, gsz):
    nblk = (n_sel + gsz - 1) // gsz

    for b in range(4):
        @pl.when(b < nblk)
        def _issue0(b=b):
            pltpu.async_copy(hp.at[selsrc.at[pl.ds(soff + b * gsz, gsz)]],
                             rows.at[b], sems[b])

    nq = (nblk + 3) // 4

    def drain_body(q, carry):
        for b in range(4):
            g = q * 4 + b

            @pl.when(g < nblk)
            def _proc(b=b, g=g):
                pltpu.make_async_copy(
                    hp.at[selsrc.at[pl.ds(soff + g * gsz, gsz)]],
                    rows.at[b], sems[b]).wait()

                def sub_body(sub, carry):
                    for i in range(16):
                        rowsel = plsc.load_gather(
                            seldst,
                            [jnp.full((16,), soff + g * gsz + sub * 16 + i,
                                      jnp.int32)])
                        abase = rowsel * _D
                        idxs = [abase + (iota + 16 * j) for j in range(8)]
                        accs = [plsc.load_gather(acc, [idxs[j]])
                                for j in range(8)]
                        rvs = [rows[b, sub * 16 + i, pl.ds(16 * j, 16)]
                               for j in range(8)]
                        news = [jnp.maximum(accs[j], rvs[j])
                                for j in range(8)]
                        for j in range(8):
                            plsc.store_scatter(acc, [idxs[j]], news[j])
                    return carry

                lax.fori_loop(0, gsz // 16, sub_body, 0)
                nxt = g + 4

                @pl.when(nxt < nblk)
                def _issue():
                    pltpu.async_copy(
                        hp.at[selsrc.at[pl.ds(soff + nxt * gsz, gsz)]],
                        rows.at[b], sems[b])

        return carry

    lax.fori_loop(0, nq, drain_body, 0)


def _segmax_full_body(hp, srcg, dstg, hn, selo, seldo, cnto,
                      acc, srcb, dstb, selsrc, seldst, rows, cntv,
                      s0, s1, s2, s3):
    sems = (s0, s1, s2, s3)
    cid = lax.axis_index("c")
    sid = lax.axis_index("s")
    wid = sid * _NC + cid
    base = wid * _R
    iota = lax.iota(jnp.int32, 16)

    _zero_acc(acc)

    def do_chunk(c, carry):
        pltpu.sync_copy(srcg.at[pl.ds(c * _C, _C)], srcb)
        pltpu.sync_copy(dstg.at[pl.ds(c * _C, _C)], dstb)

        def scan_body(g, cur):
            dstv = dstb[pl.ds(g * 16, 16)]
            srcv = srcb[pl.ds(g * 16, 16)]
            m = (dstv >= base) & (dstv < base + _R)
            dstl = dstv - base
            p = jnp.where(m, 1, 0).astype(jnp.int32)
            for k in (1, 2, 4, 8):
                sh = _take16(p, jnp.maximum(iota - k, 0))
                p = p + jnp.where(iota >= k, sh, 0)
            pos = cur + p - 1
            plsc.store_scatter(selsrc, [pos], srcv, mask=m)
            plsc.store_scatter(seldst, [pos], dstl, mask=m)
            cnt = _take16(p, jnp.full((16,), 15, jnp.int32))
            return cur + cnt

        cur = lax.fori_loop(0, _NG, scan_body, jnp.zeros((16,), jnp.int32))
        n_sel = jnp.max(cur)

        for q in range(5):
            posp = n_sel + iota + q * 16
            padsrc = iota * 577 + wid * 29 + q * 89
            plsc.store_scatter(selsrc, [posp], padsrc)
            plsc.store_scatter(seldst, [posp], jnp.full((16,), _R, jnp.int32))

        plsc.store_scatter(cntv, [jnp.full((16,), c, jnp.int32)],
                           jnp.full((16,), n_sel, jnp.int32))

        _emit_drain(hp, selsrc, seldst, 0, n_sel, acc, rows, sems, iota, _G)

        slot = (wid * _NCHUNK + c) * _SELSZ
        for t in range(5):
            @pl.when(t * _SBLK < n_sel + 80)
            def _spill(t=t):
                pltpu.sync_copy(selsrc.at[pl.ds(t * _SBLK, _SBLK)],
                                selo.at[pl.ds(slot + t * _SBLK, _SBLK)])
                pltpu.sync_copy(seldst.at[pl.ds(t * _SBLK, _SBLK)],
                                seldo.at[pl.ds(slot + t * _SBLK, _SBLK)])
        return carry

    lax.fori_loop(0, _NCHUNK, do_chunk, 0)

    pltpu.sync_copy(cntv, cnto.at[pl.ds(wid * 32, 32)])
    pltpu.sync_copy(acc.at[pl.ds(0, _R * _D)], hn.at[pl.ds(base * _D, _R * _D)])


def _segmax_drain_body(hp, seli, seldi, cnti, hn,
                       acc, selsrc, seldst, rows, cntv,
                       s0, s1, s2, s3, p0, p1, p2, p3):
    sems = (s0, s1, s2, s3)
    psems = ((p0, p1), (p2, p3))
    cid = lax.axis_index("c")
    sid = lax.axis_index("s")
    wid = sid * _NC + cid
    base = wid * _R
    iota = lax.iota(jnp.int32, 16)

    _zero_acc(acc)
    pltpu.sync_copy(cnti.at[pl.ds(wid * 32, 32)], cntv)

    def slot_of(c):
        return (wid * _NCHUNK + c) * _SELSZ

    def issue_b0(c, h):
        slot = slot_of(c)
        pltpu.async_copy(seli.at[pl.ds(slot, _SBLK)],
                         selsrc.at[pl.ds(h * _SELSZ, _SBLK)], psems[h][0])
        pltpu.async_copy(seldi.at[pl.ds(slot, _SBLK)],
                         seldst.at[pl.ds(h * _SELSZ, _SBLK)], psems[h][1])

    def wait_b0(c, h):
        slot = slot_of(c)
        pltpu.make_async_copy(seli.at[pl.ds(slot, _SBLK)],
                              selsrc.at[pl.ds(h * _SELSZ, _SBLK)],
                              psems[h][0]).wait()
        pltpu.make_async_copy(seldi.at[pl.ds(slot, _SBLK)],
                              seldst.at[pl.ds(h * _SELSZ, _SBLK)],
                              psems[h][1]).wait()

    def do_chunk(c, h):
        cv = plsc.load_gather(cntv, [jnp.full((16,), c, jnp.int32)])
        n_sel = jnp.max(cv)
        wait_b0(c, h)
        slot = slot_of(c)
        for t in range(1, 5):
            @pl.when(t * _SBLK < n_sel + 80)
            def _load(t=t):
                pltpu.sync_copy(seli.at[pl.ds(slot + t * _SBLK, _SBLK)],
                                selsrc.at[pl.ds(h * _SELSZ + t * _SBLK, _SBLK)])
                pltpu.sync_copy(seldi.at[pl.ds(slot + t * _SBLK, _SBLK)],
                                seldst.at[pl.ds(h * _SELSZ + t * _SBLK, _SBLK)])

        _emit_drain(hp, selsrc, seldst, h * _SELSZ, n_sel, acc, rows, sems,
                    iota, 2 * _G)

    issue_b0(0, 0)
    issue_b0(1, 1)

    def pair_body(q, carry):
        c0 = 2 * q
        do_chunk(c0, 0)
        issue_b0(c0 + 2, 0)
        do_chunk(c0 + 1, 1)

        @pl.when(c0 + 3 < _NCHUNK)
        def _pf():
            issue_b0(c0 + 3, 1)

        return carry

    lax.fori_loop(0, (_NCHUNK - 1) // 2, pair_body, 0)
    do_chunk(_NCHUNK - 1, 0)

    pltpu.sync_copy(acc.at[pl.ds(0, _R * _D)], hn.at[pl.ds(base * _D, _R * _D)])


_sc_mesh = plsc.VectorSubcoreMesh(core_axis_name="c", subcore_axis_name="s",
                                  num_cores=_NC, num_subcores=_NS)
_sc_params = pltpu.CompilerParams(needs_layout_passes=False)

_segmax_full = pl.kernel(
    _segmax_full_body,
    out_type=(
        jax.ShapeDtypeStruct((_NPAD * _D,), jnp.float32),
        jax.ShapeDtypeStruct((_NW * _NCHUNK * _SELSZ,), jnp.int32),
        jax.ShapeDtypeStruct((_NW * _NCHUNK * _SELSZ,), jnp.int32),
        jax.ShapeDtypeStruct((_NW * 32,), jnp.int32),
    ),
    mesh=_sc_mesh,
    compiler_params=_sc_params,
    scratch_types=[
        pltpu.VMEM(((_R + 1) * _D,), jnp.float32),
        pltpu.VMEM((_C,), jnp.int32),
        pltpu.VMEM((_C,), jnp.int32),
        pltpu.VMEM((_SELSZ,), jnp.int32),
        pltpu.VMEM((_SELSZ,), jnp.int32),
        pltpu.VMEM((4, _G, _D), jnp.float32),
        pltpu.VMEM((32,), jnp.int32),
        pltpu.SemaphoreType.DMA, pltpu.SemaphoreType.DMA,
        pltpu.SemaphoreType.DMA, pltpu.SemaphoreType.DMA,
    ],
)

_segmax_drain = pl.kernel(
    _segmax_drain_body,
    out_type=jax.ShapeDtypeStruct((_NPAD * _D,), jnp.float32),
    mesh=_sc_mesh,
    compiler_params=_sc_params,
    scratch_types=[
        pltpu.VMEM(((_R + 1) * _D,), jnp.float32),
        pltpu.VMEM((2 * _SELSZ,), jnp.int32),
        pltpu.VMEM((2 * _SELSZ,), jnp.int32),
        pltpu.VMEM((4, 2 * _G, _D), jnp.float32),
        pltpu.VMEM((32,), jnp.int32),
        pltpu.SemaphoreType.DMA, pltpu.SemaphoreType.DMA,
        pltpu.SemaphoreType.DMA, pltpu.SemaphoreType.DMA,
        pltpu.SemaphoreType.DMA, pltpu.SemaphoreType.DMA,
        pltpu.SemaphoreType.DMA, pltpu.SemaphoreType.DMA,
    ],
)


def kernel(h, edge_index, he, W_pool0, b_pool0, W_self0, W_neigh0, b_neigh0,
           W_pool1, b_pool1, W_self1, W_neigh1, b_neigh1, W_lin, b_lin):
    src = edge_index[0]
    dst = edge_index[1]

    hp0 = _mm_relu(h, W_pool0.T, b_pool0)
    hn0, selo, seldo, cnts = _segmax_full(hp0, src, dst)
    hn0 = hn0.reshape(_NPAD, _D)[:_N]
    h1 = _combine_relu(h, hn0, W_self0.T, W_neigh0.T, b_neigh0)

    hp1 = _mm_relu(h1, W_pool1.T, b_pool1)
    hn1 = _segmax_drain(hp1, selo, seldo, cnts).reshape(_NPAD, _D)[:_N]
    h2 = _combine_relu(h1, hn1, W_self1.T, W_neigh1.T, b_neigh1)

    global_feat = _mean_head(h2, W_lin.T, b_lin)
    return (h2, global_feat)


# speedup vs baseline: 3.7465x; 1.2590x over previous
import jax
import jax.numpy as jnp
from jax import lax
from jax.experimental import pallas as pl
from jax.experimental.pallas import tpu as pltpu
from jax.experimental.pallas import tpu_sc as plsc

_N = 10000
_D = 128
_E = 320000
_BLK = 1000

_NC = 2
_NS = 16
_NW = _NC * _NS
_R = 320
_NPAD = _NW * _R
_C = 12800
_NG = _C // 16
_NCHUNK = _E // _C
_SELSZ = _C + 80
_SBLK = 2576
_G = 32


def _mm_relu(x, wt, b):
    def body(x_ref, w_ref, b_ref, o_ref):
        acc = jnp.dot(x_ref[...], w_ref[...], preferred_element_type=jnp.float32)
        o_ref[...] = jax.nn.relu(acc + b_ref[...])

    n = x.shape[0]
    return pl.pallas_call(
        body,
        grid=(n // _BLK,),
        in_specs=[
            pl.BlockSpec((_BLK, _D), lambda i: (i, 0)),
            pl.BlockSpec((_D, _D), lambda i: (0, 0)),
            pl.BlockSpec((1, _D), lambda i: (0, 0)),
        ],
        out_specs=pl.BlockSpec((_BLK, _D), lambda i: (i, 0)),
        out_shape=jax.ShapeDtypeStruct((n, _D), jnp.float32),
    )(x, wt, b.reshape(1, _D))


def _combine_relu(h, hn, wst, wnt, b):
    def body(h_ref, hn_ref, ws_ref, wn_ref, b_ref, o_ref):
        acc = jnp.dot(h_ref[...], ws_ref[...], preferred_element_type=jnp.float32)
        acc += jnp.dot(hn_ref[...], wn_ref[...], preferred_element_type=jnp.float32)
        o_ref[...] = jax.nn.relu(acc + b_ref[...])

    return pl.pallas_call(
        body,
        grid=(_N // _BLK,),
        in_specs=[
            pl.BlockSpec((_BLK, _D), lambda i: (i, 0)),
            pl.BlockSpec((_BLK, _D), lambda i: (i, 0)),
            pl.BlockSpec((_D, _D), lambda i: (0, 0)),
            pl.BlockSpec((_D, _D), lambda i: (0, 0)),
            pl.BlockSpec((1, _D), lambda i: (0, 0)),
        ],
        out_specs=pl.BlockSpec((_BLK, _D), lambda i: (i, 0)),
        out_shape=jax.ShapeDtypeStruct((_N, _D), jnp.float32),
    )(h, hn, wst, wnt, b.reshape(1, _D))


def _mean_head(h, wlt, b):
    def body(h_ref, w_ref, b_ref, o_ref, acc_ref):
        i = pl.program_id(0)

        @pl.when(i == 0)
        def _():
            acc_ref[...] = jnp.zeros_like(acc_ref)

        acc_ref[...] += jnp.sum(h_ref[...], axis=0, keepdims=True)

        @pl.when(i == _N // _BLK - 1)
        def _():
            pooled = acc_ref[...] * (1.0 / _N)
            o_ref[...] = (
                jnp.dot(pooled, w_ref[...], preferred_element_type=jnp.float32)
                + b_ref[...]
            )

    return pl.pallas_call(
        body,
        grid=(_N // _BLK,),
        in_specs=[
            pl.BlockSpec((_BLK, _D), lambda i: (i, 0)),
            pl.BlockSpec((_D, _D), lambda i: (0, 0)),
            pl.BlockSpec((1, _D), lambda i: (0, 0)),
        ],
        out_specs=pl.BlockSpec((1, _D), lambda i: (0, 0)),
        out_shape=jax.ShapeDtypeStruct((1, _D), jnp.float32),
        scratch_shapes=[pltpu.VMEM((1, _D), jnp.float32)],
    )(h, wlt, b.reshape(1, _D))


def _take16(x, idx):
    dn = lax.GatherDimensionNumbers(offset_dims=(), collapsed_slice_dims=(0,),
                                    start_index_map=(0,))
    return lax.gather(x, idx.reshape(16, 1), dn, (1,),
                      mode=lax.GatherScatterMode.PROMISE_IN_BOUNDS)


def _zero_acc(acc):
    zeros16 = jnp.zeros((16,), jnp.float32)

    def zero_body(r, carry):
        acc[pl.ds(r * 16, 16)] = zeros16
        return carry

    lax.fori_loop(0, (_R + 1) * _D // 16, zero_body, 0)


def _emit_drain(hp, selsrc, seldst, soff, n_sel, acc, rows, sems, iota, gsz):
    nblk = (n_sel + gsz - 1) // gsz

    for b in range(4):
        @pl.when(b < nblk)
        def _issue0(b=b):
            pltpu.async_copy(hp.at[selsrc.at[pl.ds(soff + b * gsz, gsz)]],
                             rows.at[b], sems[b])

    nq = (nblk + 3) // 4

    def drain_body(q, carry):
        for b in range(4):
            g = q * 4 + b

            @pl.when(g < nblk)
            def _proc(b=b, g=g):
                pltpu.make_async_copy(
                    hp.at[selsrc.at[pl.ds(soff + g * gsz, gsz)]],
                    rows.at[b], sems[b]).wait()

                def sub_body(sub, carry):
                    for i in range(16):
                        rowsel = plsc.load_gather(
                            seldst,
                            [jnp.full((16,), soff + g * gsz + sub * 16 + i,
                                      jnp.int32)])
                        abase = rowsel * _D
                        idxs = [abase + (iota + 16 * j) for j in range(8)]
                        accs = [plsc.load_gather(acc, [idxs[j]])
                                for j in range(8)]
                        rvs = [rows[b, sub * 16 + i, pl.ds(16 * j, 16)]
                               for j in range(8)]
                        news = [jnp.maximum(accs[j], rvs[j])
                                for j in range(8)]
                        for j in range(8):
                            plsc.store_scatter(acc, [idxs[j]], news[j])
                    return carry

                lax.fori_loop(0, gsz // 16, sub_body, 0)
                nxt = g + 4

                @pl.when(nxt < nblk)
                def _issue():
                    pltpu.async_copy(
                        hp.at[selsrc.at[pl.ds(soff + nxt * gsz, gsz)]],
                        rows.at[b], sems[b])

        return carry

    lax.fori_loop(0, nq, drain_body, 0)


def _segmax_full_body(hp, srcg, dstg, hn, selo, seldo, cnto,
                      acc, srcb, dstb, selsrc, seldst, rows, cntv,
                      s0, s1, s2, s3):
    sems = (s0, s1, s2, s3)
    cid = lax.axis_index("c")
    sid = lax.axis_index("s")
    wid = sid * _NC + cid
    base = wid * _R
    iota = lax.iota(jnp.int32, 16)

    _zero_acc(acc)

    def do_chunk(c, carry):
        pltpu.sync_copy(srcg.at[pl.ds(c * _C, _C)], srcb)
        pltpu.sync_copy(dstg.at[pl.ds(c * _C, _C)], dstb)

        def scan_body(it, cur):
            ms, dstls, srcvs, ps = [], [], [], []
            for u in range(4):
                dstv = dstb[pl.ds(it * 64 + u * 16, 16)]
                srcv = srcb[pl.ds(it * 64 + u * 16, 16)]
                m = (dstv >= base) & (dstv < base + _R)
                p = jnp.where(m, 1, 0).astype(jnp.int32)
                for k in (1, 2, 4, 8):
                    sh = _take16(p, jnp.maximum(iota - k, 0))
                    p = p + jnp.where(iota >= k, sh, 0)
                ms.append(m)
                dstls.append(dstv - base)
                srcvs.append(srcv)
                ps.append(p)
            for u in range(4):
                pos = cur + ps[u] - 1
                plsc.store_scatter(selsrc, [pos], srcvs[u], mask=ms[u])
                plsc.store_scatter(seldst, [pos], dstls[u], mask=ms[u])
                cur = cur + _take16(ps[u], jnp.full((16,), 15, jnp.int32))
            return cur

        cur = lax.fori_loop(0, _NG // 4, scan_body,
                            jnp.zeros((16,), jnp.int32))
        n_sel = jnp.max(cur)

        for q in range(5):
            posp = n_sel + iota + q * 16
            padsrc = iota * 577 + wid * 29 + q * 89
            plsc.store_scatter(selsrc, [posp], padsrc)
            plsc.store_scatter(seldst, [posp], jnp.full((16,), _R, jnp.int32))

        plsc.store_scatter(cntv, [jnp.full((16,), c, jnp.int32)],
                           jnp.full((16,), n_sel, jnp.int32))

        _emit_drain(hp, selsrc, seldst, 0, n_sel, acc, rows, sems, iota, _G)

        slot = (wid * _NCHUNK + c) * _SELSZ
        for t in range(5):
            @pl.when(t * _SBLK < n_sel + 80)
            def _spill(t=t):
                pltpu.sync_copy(selsrc.at[pl.ds(t * _SBLK, _SBLK)],
                                selo.at[pl.ds(slot + t * _SBLK, _SBLK)])
                pltpu.sync_copy(seldst.at[pl.ds(t * _SBLK, _SBLK)],
                                seldo.at[pl.ds(slot + t * _SBLK, _SBLK)])
        return carry

    lax.fori_loop(0, _NCHUNK, do_chunk, 0)

    pltpu.sync_copy(cntv, cnto.at[pl.ds(wid * 32, 32)])
    pltpu.sync_copy(acc.at[pl.ds(0, _R * _D)], hn.at[pl.ds(base * _D, _R * _D)])


def _segmax_drain_body(hp, seli, seldi, cnti, hn,
                       acc, selsrc, seldst, rows, cntv,
                       s0, s1, s2, s3, p0, p1, p2, p3):
    sems = (s0, s1, s2, s3)
    psems = ((p0, p1), (p2, p3))
    cid = lax.axis_index("c")
    sid = lax.axis_index("s")
    wid = sid * _NC + cid
    base = wid * _R
    iota = lax.iota(jnp.int32, 16)

    _zero_acc(acc)
    pltpu.sync_copy(cnti.at[pl.ds(wid * 32, 32)], cntv)

    def slot_of(c):
        return (wid * _NCHUNK + c) * _SELSZ

    def issue_b0(c, h):
        slot = slot_of(c)
        pltpu.async_copy(seli.at[pl.ds(slot, _SBLK)],
                         selsrc.at[pl.ds(h * _SELSZ, _SBLK)], psems[h][0])
        pltpu.async_copy(seldi.at[pl.ds(slot, _SBLK)],
                         seldst.at[pl.ds(h * _SELSZ, _SBLK)], psems[h][1])

    def wait_b0(c, h):
        slot = slot_of(c)
        pltpu.make_async_copy(seli.at[pl.ds(slot, _SBLK)],
                              selsrc.at[pl.ds(h * _SELSZ, _SBLK)],
                              psems[h][0]).wait()
        pltpu.make_async_copy(seldi.at[pl.ds(slot, _SBLK)],
                              seldst.at[pl.ds(h * _SELSZ, _SBLK)],
                              psems[h][1]).wait()

    def do_chunk(c, h):
        cv = plsc.load_gather(cntv, [jnp.full((16,), c, jnp.int32)])
        n_sel = jnp.max(cv)
        wait_b0(c, h)
        slot = slot_of(c)
        for t in range(1, 5):
            @pl.when(t * _SBLK < n_sel + 80)
            def _load(t=t):
                pltpu.sync_copy(seli.at[pl.ds(slot + t * _SBLK, _SBLK)],
                                selsrc.at[pl.ds(h * _SELSZ + t * _SBLK, _SBLK)])
                pltpu.sync_copy(seldi.at[pl.ds(slot + t * _SBLK, _SBLK)],
                                seldst.at[pl.ds(h * _SELSZ + t * _SBLK, _SBLK)])

        _emit_drain(hp, selsrc, seldst, h * _SELSZ, n_sel, acc, rows, sems,
                    iota, 2 * _G)

    issue_b0(0, 0)
    issue_b0(1, 1)

    def pair_body(q, carry):
        c0 = 2 * q
        do_chunk(c0, 0)
        issue_b0(c0 + 2, 0)
        do_chunk(c0 + 1, 1)

        @pl.when(c0 + 3 < _NCHUNK)
        def _pf():
            issue_b0(c0 + 3, 1)

        return carry

    lax.fori_loop(0, (_NCHUNK - 1) // 2, pair_body, 0)
    do_chunk(_NCHUNK - 1, 0)

    pltpu.sync_copy(acc.at[pl.ds(0, _R * _D)], hn.at[pl.ds(base * _D, _R * _D)])


_sc_mesh = plsc.VectorSubcoreMesh(core_axis_name="c", subcore_axis_name="s",
                                  num_cores=_NC, num_subcores=_NS)
_sc_params = pltpu.CompilerParams(needs_layout_passes=False)

_segmax_full = pl.kernel(
    _segmax_full_body,
    out_type=(
        jax.ShapeDtypeStruct((_NPAD * _D,), jnp.float32),
        jax.ShapeDtypeStruct((_NW * _NCHUNK * _SELSZ,), jnp.int32),
        jax.ShapeDtypeStruct((_NW * _NCHUNK * _SELSZ,), jnp.int32),
        jax.ShapeDtypeStruct((_NW * 32,), jnp.int32),
    ),
    mesh=_sc_mesh,
    compiler_params=_sc_params,
    scratch_types=[
        pltpu.VMEM(((_R + 1) * _D,), jnp.float32),
        pltpu.VMEM((_C,), jnp.int32),
        pltpu.VMEM((_C,), jnp.int32),
        pltpu.VMEM((_SELSZ,), jnp.int32),
        pltpu.VMEM((_SELSZ,), jnp.int32),
        pltpu.VMEM((4, _G, _D), jnp.float32),
        pltpu.VMEM((32,), jnp.int32),
        pltpu.SemaphoreType.DMA, pltpu.SemaphoreType.DMA,
        pltpu.SemaphoreType.DMA, pltpu.SemaphoreType.DMA,
    ],
)

_segmax_drain = pl.kernel(
    _segmax_drain_body,
    out_type=jax.ShapeDtypeStruct((_NPAD * _D,), jnp.float32),
    mesh=_sc_mesh,
    compiler_params=_sc_params,
    scratch_types=[
        pltpu.VMEM(((_R + 1) * _D,), jnp.float32),
        pltpu.VMEM((2 * _SELSZ,), jnp.int32),
        pltpu.VMEM((2 * _SELSZ,), jnp.int32),
        pltpu.VMEM((4, 2 * _G, _D), jnp.float32),
        pltpu.VMEM((32,), jnp.int32),
        pltpu.SemaphoreType.DMA, pltpu.SemaphoreType.DMA,
        pltpu.SemaphoreType.DMA, pltpu.SemaphoreType.DMA,
        pltpu.SemaphoreType.DMA, pltpu.SemaphoreType.DMA,
        pltpu.SemaphoreType.DMA, pltpu.SemaphoreType.DMA,
    ],
)


def kernel(h, edge_index, he, W_pool0, b_pool0, W_self0, W_neigh0, b_neigh0,
           W_pool1, b_pool1, W_self1, W_neigh1, b_neigh1, W_lin, b_lin):
    src = edge_index[0]
    dst = edge_index[1]

    hp0 = _mm_relu(h, W_pool0.T, b_pool0)
    hn0, selo, seldo, cnts = _segmax_full(hp0, src, dst)
    hn0 = hn0.reshape(_NPAD, _D)[:_N]
    h1 = _combine_relu(h, hn0, W_self0.T, W_neigh0.T, b_neigh0)

    hp1 = _mm_relu(h1, W_pool1.T, b_pool1)
    hn1 = _segmax_drain(hp1, selo, seldo, cnts).reshape(_NPAD, _D)[:_N]
    h2 = _combine_relu(h1, hn1, W_self1.T, W_neigh1.T, b_neigh1)

    global_feat = _mean_head(h2, W_lin.T, b_lin)
    return (h2, global_feat)
